# Initial kernel scaffold; baseline (speedup 1.0000x reference)
import jax
import jax.numpy as jnp
from jax.experimental import pallas as pl


def kernel(x, edge_index, edge_attr, atom_tables, bond_tables, W_input, W_hidden, W_output, b_output, bn_gamma, bn_beta):
    raise NotImplementedError("write your pallas kernel here")


# speedup vs baseline: 1.0715x; 1.0715x over previous
import functools
import jax
import jax.numpy as jnp
from jax import lax
from jax.experimental import pallas as pl
from jax.experimental.pallas import tpu as pltpu
from jax.experimental.pallas import tpu_sc as plsc

D = 128
N_NODES = 10000
N_EDGES = 320000
E2 = N_EDGES // 2
N_LAYERS = 5
EPS = 1e-5


def _encode_body(x_ref, at_ref, wt_ref, h_ref, hw_ref):
    xb = x_ref[...]
    acc = jnp.zeros((x_ref.shape[0], D), jnp.float32)
    for f in range(at_ref.shape[0]):
        oh = (xb[:, f][:, None] == lax.broadcasted_iota(
            jnp.int32, (x_ref.shape[0], at_ref.shape[1]), 1)).astype(jnp.float32)
        acc = acc + jnp.dot(oh, at_ref[f], preferred_element_type=jnp.float32)
    h_ref[...] = acc
    hw_ref[...] = jnp.dot(acc, wt_ref[...], preferred_element_type=jnp.float32)


def tc_encode(x, atom_tables, w_top):
    Bn = 2000
    grid = (N_NODES // Bn,)
    return pl.pallas_call(
        _encode_body,
        grid=grid,
        in_specs=[
            pl.BlockSpec((Bn, x.shape[1]), lambda i: (i, 0)),
            pl.BlockSpec(atom_tables.shape, lambda i: (0, 0, 0)),
            pl.BlockSpec((D, D), lambda i: (0, 0)),
        ],
        out_specs=[
            pl.BlockSpec((Bn, D), lambda i: (i, 0)),
            pl.BlockSpec((Bn, D), lambda i: (i, 0)),
        ],
        out_shape=[
            jax.ShapeDtypeStruct((N_NODES, D), jnp.float32),
            jax.ShapeDtypeStruct((N_NODES, D), jnp.float32),
        ],
    )(x, atom_tables, w_top)


def _msg0_body(g_ref, ea_ref, bt_ref, wb_ref, o_ref):
    ea = ea_ref[...]
    acc = jnp.zeros((ea_ref.shape[0], D), jnp.float32)
    for f in range(bt_ref.shape[0]):
        oh = (ea[:, f][:, None] == lax.broadcasted_iota(
            jnp.int32, (ea_ref.shape[0], bt_ref.shape[1]), 1)).astype(jnp.float32)
        acc = acc + jnp.dot(oh, bt_ref[f], preferred_element_type=jnp.float32)
    o_ref[...] = g_ref[...] + jnp.dot(acc, wb_ref[...],
                                      preferred_element_type=jnp.float32)


def tc_msg0(g0, edge_attr, bond_tables, w_bot):
    Be = 4000
    grid = (N_EDGES // Be,)
    return pl.pallas_call(
        _msg0_body,
        grid=grid,
        in_specs=[
            pl.BlockSpec((Be, D), lambda i: (i, 0)),
            pl.BlockSpec((Be, edge_attr.shape[1]), lambda i: (i, 0)),
            pl.BlockSpec(bond_tables.shape, lambda i: (0, 0, 0)),
            pl.BlockSpec((D, D), lambda i: (0, 0)),
        ],
        out_specs=pl.BlockSpec((Be, D), lambda i: (i, 0)),
        out_shape=jax.ShapeDtypeStruct((N_EDGES, D), jnp.float32),
    )(g0, edge_attr, bond_tables, w_bot)


def _sum_parts_body(p_ref, o_ref):
    o_ref[...] = p_ref[0] + p_ref[1]


def tc_sum_parts(parts):
    Bn = 2000
    grid = (N_NODES // Bn,)
    return pl.pallas_call(
        _sum_parts_body,
        grid=grid,
        in_specs=[pl.BlockSpec((2, Bn, D), lambda i: (0, i, 0))],
        out_specs=pl.BlockSpec((Bn, D), lambda i: (i, 0)),
        out_shape=jax.ShapeDtypeStruct((N_NODES, D), jnp.float32),
    )(parts)


def _mm_stats_body(g_ref, m_ref, w_ref, y_ref, s_ref):
    g = g_ref[...]
    m = m_ref[...]
    m_rev = jnp.concatenate([m[:, D:], m[:, :D]], axis=1)
    u = g - m_rev
    w = w_ref[...]
    y_lo = jnp.dot(u[:, :D], w, preferred_element_type=jnp.float32)
    y_hi = jnp.dot(u[:, D:], w, preferred_element_type=jnp.float32)
    y = jnp.concatenate([y_lo, y_hi], axis=1)
    y_ref[...] = y

    @pl.when(pl.program_id(0) == 0)
    def _():
        s_ref[...] = jnp.zeros_like(s_ref)

    s = jnp.sum(y, axis=0)
    sq = jnp.sum(y * y, axis=0)
    s_ref[0, :] += s
    s_ref[1, :] += sq


def tc_matmul_stats(g256, msg256, w_hidden):
    Bp = 3200
    grid = (E2 // Bp,)
    return pl.pallas_call(
        _mm_stats_body,
        grid=grid,
        in_specs=[
            pl.BlockSpec((Bp, 2 * D), lambda i: (i, 0)),
            pl.BlockSpec((Bp, 2 * D), lambda i: (i, 0)),
            pl.BlockSpec((D, D), lambda i: (0, 0)),
        ],
        out_specs=[
            pl.BlockSpec((Bp, 2 * D), lambda i: (i, 0)),
            pl.BlockSpec((8, 2 * D), lambda i: (0, 0)),
        ],
        out_shape=[
            jax.ShapeDtypeStruct((E2, 2 * D), jnp.float32),
            jax.ShapeDtypeStruct((8, 2 * D), jnp.float32),
        ],
    )(g256, msg256, w_hidden)


def _affine_relu_body(y_ref, a_ref, b_ref, o_ref):
    o_ref[...] = jnp.maximum(y_ref[...] * a_ref[...] + b_ref[...], 0.0)


def tc_affine_relu(y256, a256, b256):
    Bp = 4000
    grid = (E2 // Bp,)
    return pl.pallas_call(
        _affine_relu_body,
        grid=grid,
        in_specs=[
            pl.BlockSpec((Bp, 2 * D), lambda i: (i, 0)),
            pl.BlockSpec((1, 2 * D), lambda i: (0, 0)),
            pl.BlockSpec((1, 2 * D), lambda i: (0, 0)),
        ],
        out_specs=pl.BlockSpec((Bp, 2 * D), lambda i: (i, 0)),
        out_shape=jax.ShapeDtypeStruct((E2, 2 * D), jnp.float32),
    )(y256, a256, b256)


def _final_body(h_ref, p_ref, wt_ref, wb_ref, b_ref, o_ref):
    nm = p_ref[0] + p_ref[1]
    r = (jnp.dot(h_ref[...], wt_ref[...], preferred_element_type=jnp.float32)
         + jnp.dot(nm, wb_ref[...], preferred_element_type=jnp.float32)
         + b_ref[...])
    o_ref[...] = jnp.maximum(r, 0.0)


def tc_final(h, parts, w_top, w_bot, bias):
    Bn = 2000
    grid = (N_NODES // Bn,)
    return pl.pallas_call(
        _final_body,
        grid=grid,
        in_specs=[
            pl.BlockSpec((Bn, D), lambda i: (i, 0)),
            pl.BlockSpec((2, Bn, D), lambda i: (0, i, 0)),
            pl.BlockSpec((D, D), lambda i: (0, 0)),
            pl.BlockSpec((D, D), lambda i: (0, 0)),
            pl.BlockSpec((1, D), lambda i: (0, 0)),
        ],
        out_specs=pl.BlockSpec((Bn, D), lambda i: (i, 0)),
        out_shape=jax.ShapeDtypeStruct((N_NODES, D), jnp.float32),
    )(h, parts, w_top, w_bot, bias)


def sc_gather(table, idx_rows):
    return jnp.take(table, idx_rows.reshape(-1), axis=0)


def sc_scatter(msg, dst_rows, zeros):
    nm = jax.ops.segment_sum(msg, dst_rows.reshape(-1), num_segments=N_NODES)
    return jnp.stack([nm, jnp.zeros_like(nm)], axis=0)


def kernel(x, edge_index, edge_attr, atom_tables, bond_tables,
           W_input, W_hidden, W_output, b_output, bn_gamma, bn_beta):
    x = x.astype(jnp.int32)
    edge_attr = edge_attr.astype(jnp.int32)
    src = edge_index[0].astype(jnp.int32)
    dst = edge_index[1].astype(jnp.int32)
    dst_rows = dst.reshape(N_EDGES // 128, 128)
    src_rows = src.reshape(N_EDGES // 128, 128)
    zeros = jnp.zeros((N_NODES, D), jnp.float32)

    h, hW = tc_encode(x, atom_tables, W_input[:D])
    g0 = sc_gather(hW, dst_rows)
    msg = tc_msg0(g0, edge_attr, bond_tables, W_input[D:])
    msg256 = msg.reshape(E2, 2 * D)

    inv_e = 1.0 / N_EDGES
    for i in range(N_LAYERS - 1):
        parts = sc_scatter(msg256.reshape(N_EDGES, D), dst_rows, zeros)
        nm = tc_sum_parts(parts)
        g = sc_gather(nm, src_rows)
        y256, stats = tc_matmul_stats(g.reshape(E2, 2 * D), msg256, W_hidden)
        s = (stats[0, :D] + stats[0, D:]) * inv_e
        sq = (stats[1, :D] + stats[1, D:]) * inv_e
        var = sq - s * s
        a = bn_gamma[i] / jnp.sqrt(var + EPS)
        b = bn_beta[i] - s * a
        a256 = jnp.concatenate([a, a])[None, :]
        b256 = jnp.concatenate([b, b])[None, :]
        msg256 = tc_affine_relu(y256, a256, b256)

    parts = sc_scatter(msg256.reshape(N_EDGES, D), dst_rows, zeros)
    return tc_final(h, parts, W_output[:D], W_output[D:], b_output[None, :])


# speedup vs baseline: 2.2873x; 2.1347x over previous
import functools
import jax
import jax.numpy as jnp
from jax import lax
from jax.experimental import pallas as pl
from jax.experimental.pallas import tpu as pltpu
from jax.experimental.pallas import tpu_sc as plsc

D = 128
N_NODES = 10000
N_EDGES = 320000
E2 = N_EDGES // 2
N_LAYERS = 5
EPS = 1e-5


def _encode_body(x_ref, at_ref, wt_ref, h_ref, hw_ref):
    xb = x_ref[...]
    acc = jnp.zeros((x_ref.shape[0], D), jnp.float32)
    for f in range(at_ref.shape[0]):
        oh = (xb[:, f][:, None] == lax.broadcasted_iota(
            jnp.int32, (x_ref.shape[0], at_ref.shape[1]), 1)).astype(jnp.float32)
        acc = acc + jnp.dot(oh, at_ref[f], preferred_element_type=jnp.float32)
    h_ref[...] = acc
    hw_ref[...] = jnp.dot(acc, wt_ref[...], preferred_element_type=jnp.float32)


def tc_encode(x, atom_tables, w_top):
    Bn = 2000
    grid = (N_NODES // Bn,)
    return pl.pallas_call(
        _encode_body,
        grid=grid,
        in_specs=[
            pl.BlockSpec((Bn, x.shape[1]), lambda i: (i, 0)),
            pl.BlockSpec(atom_tables.shape, lambda i: (0, 0, 0)),
            pl.BlockSpec((D, D), lambda i: (0, 0)),
        ],
        out_specs=[
            pl.BlockSpec((Bn, D), lambda i: (i, 0)),
            pl.BlockSpec((Bn, D), lambda i: (i, 0)),
        ],
        out_shape=[
            jax.ShapeDtypeStruct((N_NODES, D), jnp.float32),
            jax.ShapeDtypeStruct((N_NODES, D), jnp.float32),
        ],
    )(x, atom_tables, w_top)


def _msg0_body(g_ref, ea_ref, bt_ref, wb_ref, o_ref):
    ea = ea_ref[...]
    acc = jnp.zeros((ea_ref.shape[0], D), jnp.float32)
    for f in range(bt_ref.shape[0]):
        oh = (ea[:, f][:, None] == lax.broadcasted_iota(
            jnp.int32, (ea_ref.shape[0], bt_ref.shape[1]), 1)).astype(jnp.float32)
        acc = acc + jnp.dot(oh, bt_ref[f], preferred_element_type=jnp.float32)
    o_ref[...] = g_ref[...] + jnp.dot(acc, wb_ref[...],
                                      preferred_element_type=jnp.float32)


def tc_msg0(g0, edge_attr, bond_tables, w_bot):
    Be = 4000
    grid = (N_EDGES // Be,)
    return pl.pallas_call(
        _msg0_body,
        grid=grid,
        in_specs=[
            pl.BlockSpec((Be, D), lambda i: (i, 0)),
            pl.BlockSpec((Be, edge_attr.shape[1]), lambda i: (i, 0)),
            pl.BlockSpec(bond_tables.shape, lambda i: (0, 0, 0)),
            pl.BlockSpec((D, D), lambda i: (0, 0)),
        ],
        out_specs=pl.BlockSpec((Be, D), lambda i: (i, 0)),
        out_shape=jax.ShapeDtypeStruct((N_EDGES, D), jnp.float32),
    )(g0, edge_attr, bond_tables, w_bot)


def _sum_parts_body(p_ref, o_ref):
    o_ref[...] = p_ref[0] + p_ref[1]


def tc_sum_parts(parts):
    Bn = 2000
    grid = (N_NODES // Bn,)
    return pl.pallas_call(
        _sum_parts_body,
        grid=grid,
        in_specs=[pl.BlockSpec((2, Bn, D), lambda i: (0, i, 0))],
        out_specs=pl.BlockSpec((Bn, D), lambda i: (i, 0)),
        out_shape=jax.ShapeDtypeStruct((N_NODES, D), jnp.float32),
    )(parts)


def _mm_stats_body(g_ref, m_ref, w_ref, y_ref, s_ref):
    g = g_ref[...]
    m = m_ref[...]
    m_rev = jnp.concatenate([m[:, D:], m[:, :D]], axis=1)
    u = g - m_rev
    w = w_ref[...]
    y_lo = jnp.dot(u[:, :D], w, preferred_element_type=jnp.float32)
    y_hi = jnp.dot(u[:, D:], w, preferred_element_type=jnp.float32)
    y = jnp.concatenate([y_lo, y_hi], axis=1)
    y_ref[...] = y

    @pl.when(pl.program_id(0) == 0)
    def _():
        s_ref[...] = jnp.zeros_like(s_ref)

    s = jnp.sum(y, axis=0)
    sq = jnp.sum(y * y, axis=0)
    s_ref[0, :] += s
    s_ref[1, :] += sq


def tc_matmul_stats(g256, msg256, w_hidden):
    Bp = 3200
    grid = (E2 // Bp,)
    return pl.pallas_call(
        _mm_stats_body,
        grid=grid,
        in_specs=[
            pl.BlockSpec((Bp, 2 * D), lambda i: (i, 0)),
            pl.BlockSpec((Bp, 2 * D), lambda i: (i, 0)),
            pl.BlockSpec((D, D), lambda i: (0, 0)),
        ],
        out_specs=[
            pl.BlockSpec((Bp, 2 * D), lambda i: (i, 0)),
            pl.BlockSpec((8, 2 * D), lambda i: (0, 0)),
        ],
        out_shape=[
            jax.ShapeDtypeStruct((E2, 2 * D), jnp.float32),
            jax.ShapeDtypeStruct((8, 2 * D), jnp.float32),
        ],
    )(g256, msg256, w_hidden)


def _affine_relu_body(y_ref, a_ref, b_ref, o_ref):
    o_ref[...] = jnp.maximum(y_ref[...] * a_ref[...] + b_ref[...], 0.0)


def tc_affine_relu(y256, a256, b256):
    Bp = 4000
    grid = (E2 // Bp,)
    return pl.pallas_call(
        _affine_relu_body,
        grid=grid,
        in_specs=[
            pl.BlockSpec((Bp, 2 * D), lambda i: (i, 0)),
            pl.BlockSpec((1, 2 * D), lambda i: (0, 0)),
            pl.BlockSpec((1, 2 * D), lambda i: (0, 0)),
        ],
        out_specs=pl.BlockSpec((Bp, 2 * D), lambda i: (i, 0)),
        out_shape=jax.ShapeDtypeStruct((E2, 2 * D), jnp.float32),
    )(y256, a256, b256)


def _final_body(h_ref, p_ref, wt_ref, wb_ref, b_ref, o_ref):
    nm = p_ref[0] + p_ref[1]
    r = (jnp.dot(h_ref[...], wt_ref[...], preferred_element_type=jnp.float32)
         + jnp.dot(nm, wb_ref[...], preferred_element_type=jnp.float32)
         + b_ref[...])
    o_ref[...] = jnp.maximum(r, 0.0)


def tc_final(h, parts, w_top, w_bot, bias):
    Bn = 2000
    grid = (N_NODES // Bn,)
    return pl.pallas_call(
        _final_body,
        grid=grid,
        in_specs=[
            pl.BlockSpec((Bn, D), lambda i: (i, 0)),
            pl.BlockSpec((2, Bn, D), lambda i: (0, i, 0)),
            pl.BlockSpec((D, D), lambda i: (0, 0)),
            pl.BlockSpec((D, D), lambda i: (0, 0)),
            pl.BlockSpec((1, D), lambda i: (0, 0)),
        ],
        out_specs=pl.BlockSpec((Bn, D), lambda i: (i, 0)),
        out_shape=jax.ShapeDtypeStruct((N_NODES, D), jnp.float32),
    )(h, parts, w_top, w_bot, bias)


NC = 2
NS = 16
NW = NC * NS
IDX_ROWS = N_EDGES // 128
NPAD = 10112


def sc_gather(table, idx_rows):
    mesh = plsc.VectorSubcoreMesh(core_axis_name="c", subcore_axis_name="s")

    @functools.partial(
        pl.kernel, mesh=mesh,
        out_type=jax.ShapeDtypeStruct((N_EDGES, D), jnp.float32),
        scratch_types=[
            pltpu.VMEM((1, 128), jnp.int32),
            pltpu.VMEM((128, D), jnp.float32),
            pltpu.SemaphoreType.DMA,
        ],
    )
    def k(table_hbm, idx_hbm, out_hbm, idx_v, rows_v, sem):
        c = lax.axis_index("c")
        s = lax.axis_index("s")
        wid = s * NC + c

        @pl.loop(0, (IDX_ROWS + NW - 1) // NW)
        def _(t):
            j = wid + t * NW

            @pl.when(j < IDX_ROWS)
            def _():
                pltpu.sync_copy(idx_hbm.at[pl.ds(j, 1)], idx_v)
                pltpu.async_copy(table_hbm.at[idx_v.at[0]], rows_v, sem).wait()
                pltpu.sync_copy(rows_v, out_hbm.at[pl.ds(j * 128, 128)])

    return k(table, idx_rows)


def sc_scatter(msg, dst_rows, zeros):
    mesh = plsc.VectorSubcoreMesh(core_axis_name="c", subcore_axis_name="s")
    rows_per_core = IDX_ROWS // NC
    stripe = NPAD // NS

    @functools.partial(
        pl.kernel, mesh=mesh,
        out_type=jax.ShapeDtypeStruct((NC, NPAD, D), jnp.float32),
        scratch_types=[
            pltpu.VMEM((1, 128), jnp.int32),
            pltpu.VMEM((128, D), jnp.float32),
            pltpu.VMEM_SHARED((NPAD, D), jnp.float32),
            pltpu.SemaphoreType.DMA,
        ],
    )
    def k(msg_hbm, idx_hbm, z_hbm, out_hbm, idx_v, rows_v, acc_sh, sem):
        c = lax.axis_index("c")
        s = lax.axis_index("s")
        pltpu.sync_copy(z_hbm.at[pl.ds(s * stripe, stripe)],
                        acc_sh.at[pl.ds(s * stripe, stripe)])
        plsc.subcore_barrier()

        @pl.loop(0, (rows_per_core + NS - 1) // NS)
        def _(t):
            j = c * rows_per_core + s + t * NS

            @pl.when(j < (c + 1) * rows_per_core)
            def _():
                pltpu.sync_copy(idx_hbm.at[pl.ds(j, 1)], idx_v)
                pltpu.sync_copy(msg_hbm.at[pl.ds(j * 128, 128)], rows_v)
                pltpu.sync_copy(rows_v, acc_sh.at[idx_v.at[0]], add=True)

        plsc.subcore_barrier()
        pltpu.sync_copy(acc_sh.at[pl.ds(s * stripe, stripe)],
                        out_hbm.at[c, pl.ds(s * stripe, stripe)])

    return k(msg, dst_rows, zeros)


def kernel(x, edge_index, edge_attr, atom_tables, bond_tables,
           W_input, W_hidden, W_output, b_output, bn_gamma, bn_beta):
    x = x.astype(jnp.int32)
    edge_attr = edge_attr.astype(jnp.int32)
    src = edge_index[0].astype(jnp.int32)
    dst = edge_index[1].astype(jnp.int32)
    dst_rows = dst.reshape(N_EDGES // 128, 128)
    src_rows = src.reshape(N_EDGES // 128, 128)
    zeros = jnp.zeros((NPAD, D), jnp.float32)

    h, hW = tc_encode(x, atom_tables, W_input[:D])
    g0 = sc_gather(hW, dst_rows)
    msg = tc_msg0(g0, edge_attr, bond_tables, W_input[D:])
    msg256 = msg.reshape(E2, 2 * D)

    inv_e = 1.0 / N_EDGES
    for i in range(N_LAYERS - 1):
        parts = sc_scatter(msg256.reshape(N_EDGES, D), dst_rows, zeros)
        nm = tc_sum_parts(parts)
        g = sc_gather(nm, src_rows)
        y256, stats = tc_matmul_stats(g.reshape(E2, 2 * D), msg256, W_hidden)
        s = (stats[0, :D] + stats[0, D:]) * inv_e
        sq = (stats[1, :D] + stats[1, D:]) * inv_e
        var = sq - s * s
        a = bn_gamma[i] / jnp.sqrt(var + EPS)
        b = bn_beta[i] - s * a
        a256 = jnp.concatenate([a, a])[None, :]
        b256 = jnp.concatenate([b, b])[None, :]
        msg256 = tc_affine_relu(y256, a256, b256)

    parts = sc_scatter(msg256.reshape(N_EDGES, D), dst_rows, zeros)
    return tc_final(h, parts, W_output[:D], W_output[D:], b_output[None, :])


# speedup vs baseline: 2.7834x; 1.2169x over previous
import functools
import jax
import jax.numpy as jnp
from jax import lax
from jax.experimental import pallas as pl
from jax.experimental.pallas import tpu as pltpu
from jax.experimental.pallas import tpu_sc as plsc

D = 128
N_NODES = 10000
N_EDGES = 320000
E2 = N_EDGES // 2
N_LAYERS = 5
EPS = 1e-5


def _encode_body(x_ref, at_ref, wt_ref, h_ref, hw_ref):
    xb = x_ref[...]
    acc = jnp.zeros((x_ref.shape[0], D), jnp.float32)
    for f in range(at_ref.shape[0]):
        oh = (xb[:, f][:, None] == lax.broadcasted_iota(
            jnp.int32, (x_ref.shape[0], at_ref.shape[1]), 1)).astype(jnp.float32)
        acc = acc + jnp.dot(oh, at_ref[f], preferred_element_type=jnp.float32)
    h_ref[...] = acc
    hw_ref[...] = jnp.dot(acc, wt_ref[...], preferred_element_type=jnp.float32)


def tc_encode(x, atom_tables, w_top):
    Bn = 2000
    grid = (N_NODES // Bn,)
    return pl.pallas_call(
        _encode_body,
        grid=grid,
        in_specs=[
            pl.BlockSpec((Bn, x.shape[1]), lambda i: (i, 0)),
            pl.BlockSpec(atom_tables.shape, lambda i: (0, 0, 0)),
            pl.BlockSpec((D, D), lambda i: (0, 0)),
        ],
        out_specs=[
            pl.BlockSpec((Bn, D), lambda i: (i, 0)),
            pl.BlockSpec((Bn, D), lambda i: (i, 0)),
        ],
        out_shape=[
            jax.ShapeDtypeStruct((N_NODES, D), jnp.float32),
            jax.ShapeDtypeStruct((N_NODES, D), jnp.float32),
        ],
    )(x, atom_tables, w_top)


def _msg0_body(g_ref, ea_ref, bt_ref, wb_ref, o_ref):
    ea = ea_ref[...]
    acc = jnp.zeros((ea_ref.shape[0], D), jnp.float32)
    for f in range(bt_ref.shape[0]):
        oh = (ea[:, f][:, None] == lax.broadcasted_iota(
            jnp.int32, (ea_ref.shape[0], bt_ref.shape[1]), 1)).astype(jnp.float32)
        acc = acc + jnp.dot(oh, bt_ref[f], preferred_element_type=jnp.float32)
    o_ref[...] = g_ref[...] + jnp.dot(acc, wb_ref[...],
                                      preferred_element_type=jnp.float32)


def tc_msg0(g0, edge_attr, bond_tables, w_bot):
    Be = 4000
    grid = (N_EDGES // Be,)
    return pl.pallas_call(
        _msg0_body,
        grid=grid,
        in_specs=[
            pl.BlockSpec((Be, D), lambda i: (i, 0)),
            pl.BlockSpec((Be, edge_attr.shape[1]), lambda i: (i, 0)),
            pl.BlockSpec(bond_tables.shape, lambda i: (0, 0, 0)),
            pl.BlockSpec((D, D), lambda i: (0, 0)),
        ],
        out_specs=pl.BlockSpec((Be, D), lambda i: (i, 0)),
        out_shape=jax.ShapeDtypeStruct((N_EDGES, D), jnp.float32),
    )(g0, edge_attr, bond_tables, w_bot)


def _sum_parts_body(p_ref, o_ref):
    o_ref[...] = p_ref[0] + p_ref[1]


def tc_sum_parts(parts):
    Bn = 2000
    grid = (N_NODES // Bn,)
    return pl.pallas_call(
        _sum_parts_body,
        grid=grid,
        in_specs=[pl.BlockSpec((2, Bn, D), lambda i: (0, i, 0))],
        out_specs=pl.BlockSpec((Bn, D), lambda i: (i, 0)),
        out_shape=jax.ShapeDtypeStruct((N_NODES, D), jnp.float32),
    )(parts)


def _mm_stats_body(g_ref, m_ref, w_ref, y_ref, s_ref):
    g = g_ref[...]
    m = m_ref[...]
    m_rev = jnp.concatenate([m[:, D:], m[:, :D]], axis=1)
    u = g - m_rev
    w = w_ref[...]
    y_lo = jnp.dot(u[:, :D], w, preferred_element_type=jnp.float32)
    y_hi = jnp.dot(u[:, D:], w, preferred_element_type=jnp.float32)
    y = jnp.concatenate([y_lo, y_hi], axis=1)
    y_ref[...] = y

    @pl.when(pl.program_id(0) == 0)
    def _():
        s_ref[...] = jnp.zeros_like(s_ref)

    s = jnp.sum(y, axis=0)
    sq = jnp.sum(y * y, axis=0)
    s_ref[0, :] += s
    s_ref[1, :] += sq


def tc_matmul_stats(g256, msg256, w_hidden):
    Bp = 3200
    grid = (E2 // Bp,)
    return pl.pallas_call(
        _mm_stats_body,
        grid=grid,
        in_specs=[
            pl.BlockSpec((Bp, 2 * D), lambda i: (i, 0)),
            pl.BlockSpec((Bp, 2 * D), lambda i: (i, 0)),
            pl.BlockSpec((D, D), lambda i: (0, 0)),
        ],
        out_specs=[
            pl.BlockSpec((Bp, 2 * D), lambda i: (i, 0)),
            pl.BlockSpec((8, 2 * D), lambda i: (0, 0)),
        ],
        out_shape=[
            jax.ShapeDtypeStruct((E2, 2 * D), jnp.float32),
            jax.ShapeDtypeStruct((8, 2 * D), jnp.float32),
        ],
    )(g256, msg256, w_hidden)


def _affine_relu_body(y_ref, a_ref, b_ref, o_ref):
    o_ref[...] = jnp.maximum(y_ref[...] * a_ref[...] + b_ref[...], 0.0)


def tc_affine_relu(y256, a256, b256):
    Bp = 4000
    grid = (E2 // Bp,)
    return pl.pallas_call(
        _affine_relu_body,
        grid=grid,
        in_specs=[
            pl.BlockSpec((Bp, 2 * D), lambda i: (i, 0)),
            pl.BlockSpec((1, 2 * D), lambda i: (0, 0)),
            pl.BlockSpec((1, 2 * D), lambda i: (0, 0)),
        ],
        out_specs=pl.BlockSpec((Bp, 2 * D), lambda i: (i, 0)),
        out_shape=jax.ShapeDtypeStruct((E2, 2 * D), jnp.float32),
    )(y256, a256, b256)


def _final_body(h_ref, p_ref, wt_ref, wb_ref, b_ref, o_ref):
    nm = p_ref[0] + p_ref[1]
    r = (jnp.dot(h_ref[...], wt_ref[...], preferred_element_type=jnp.float32)
         + jnp.dot(nm, wb_ref[...], preferred_element_type=jnp.float32)
         + b_ref[...])
    o_ref[...] = jnp.maximum(r, 0.0)


def tc_final(h, parts, w_top, w_bot, bias):
    Bn = 2000
    grid = (N_NODES // Bn,)
    return pl.pallas_call(
        _final_body,
        grid=grid,
        in_specs=[
            pl.BlockSpec((Bn, D), lambda i: (i, 0)),
            pl.BlockSpec((2, Bn, D), lambda i: (0, i, 0)),
            pl.BlockSpec((D, D), lambda i: (0, 0)),
            pl.BlockSpec((D, D), lambda i: (0, 0)),
            pl.BlockSpec((1, D), lambda i: (0, 0)),
        ],
        out_specs=pl.BlockSpec((Bn, D), lambda i: (i, 0)),
        out_shape=jax.ShapeDtypeStruct((N_NODES, D), jnp.float32),
    )(h, parts, w_top, w_bot, bias)


NC = 2
NS = 16
NW = NC * NS
IDX_ROWS = N_EDGES // 128
CPW = 80
IDX_PAD = NW * CPW
NPAD = 10112


def sc_gather(table, idx_rows):
    mesh = plsc.VectorSubcoreMesh(core_axis_name="c", subcore_axis_name="s")

    @functools.partial(
        pl.kernel, mesh=mesh,
        out_type=jax.ShapeDtypeStruct((N_EDGES, D), jnp.float32),
        scratch_types=[
            pltpu.VMEM((CPW, 128), jnp.int32),
            pltpu.VMEM((2, 128, D), jnp.float32),
            pltpu.SemaphoreType.DMA((2,)),
        ],
    )
    def k(table_hbm, idx_hbm, out_hbm, idx_v, rows_v, gsem):
        c = lax.axis_index("c")
        s = lax.axis_index("s")
        wid = s * NC + c
        lo = wid * CPW
        pltpu.sync_copy(idx_hbm.at[pl.ds(lo, CPW)], idx_v)

        def start(t, b):
            @pl.when(lo + t < IDX_ROWS)
            def _():
                pltpu.async_copy(table_hbm.at[idx_v.at[t]], rows_v.at[b],
                                 gsem.at[b])

        def finish(t, b):
            @pl.when(lo + t < IDX_ROWS)
            def _():
                pltpu.make_async_copy(table_hbm.at[idx_v.at[t]], rows_v.at[b],
                                      gsem.at[b]).wait()
                pltpu.sync_copy(rows_v.at[b], out_hbm.at[pl.ds((lo + t) * 128, 128)])

        start(0, 0)
        start(1, 1)

        @pl.loop(0, CPW // 2)
        def _(i):
            t = i * 2
            finish(t, 0)

            @pl.when(t + 2 < CPW)
            def _():
                start(t + 2, 0)

            finish(t + 1, 1)

            @pl.when(t + 3 < CPW)
            def _():
                start(t + 3, 1)

    return k(table, idx_rows)


def sc_scatter(msg, scat_idx, zeros):
    mesh = plsc.VectorSubcoreMesh(core_axis_name="c", subcore_axis_name="s")
    rows_per_core = IDX_ROWS // NC
    core_pad = NS * CPW
    stripe = NPAD // NS

    @functools.partial(
        pl.kernel, mesh=mesh,
        out_type=jax.ShapeDtypeStruct((NC, NPAD, D), jnp.float32),
        scratch_types=[
            pltpu.VMEM((CPW, 128), jnp.int32),
            pltpu.VMEM((2, 128, D), jnp.float32),
            pltpu.VMEM_SHARED((NPAD, D), jnp.float32),
            pltpu.SemaphoreType.DMA((2,)),
        ],
    )
    def k(msg_hbm, idx_hbm, z_hbm, out_hbm, idx_v, rows_v, acc_sh, lsem):
        c = lax.axis_index("c")
        s = lax.axis_index("s")
        pltpu.sync_copy(z_hbm.at[pl.ds(s * stripe, stripe)],
                        acc_sh.at[pl.ds(s * stripe, stripe)])
        base_l = s * CPW
        pltpu.sync_copy(idx_hbm.at[pl.ds(c * core_pad + base_l, CPW)], idx_v)
        plsc.subcore_barrier()

        def start(u, b):
            @pl.when(base_l + u < rows_per_core)
            def _():
                pltpu.async_copy(
                    msg_hbm.at[pl.ds((c * rows_per_core + base_l + u) * 128, 128)],
                    rows_v.at[b], lsem.at[b])

        def finish(u, b):
            @pl.when(base_l + u < rows_per_core)
            def _():
                pltpu.make_async_copy(
                    msg_hbm.at[pl.ds((c * rows_per_core + base_l + u) * 128, 128)],
                    rows_v.at[b], lsem.at[b]).wait()
                pltpu.sync_copy(rows_v.at[b], acc_sh.at[idx_v.at[u]], add=True)

        start(0, 0)
        start(1, 1)

        @pl.loop(0, CPW // 2)
        def _(i):
            u = i * 2
            finish(u, 0)

            @pl.when(u + 2 < CPW)
            def _():
                start(u + 2, 0)

            finish(u + 1, 1)

            @pl.when(u + 3 < CPW)
            def _():
                start(u + 3, 1)

        plsc.subcore_barrier()
        pltpu.sync_copy(acc_sh.at[pl.ds(s * stripe, stripe)],
                        out_hbm.at[c, pl.ds(s * stripe, stripe)])

    return k(msg, scat_idx, zeros)


def kernel(x, edge_index, edge_attr, atom_tables, bond_tables,
           W_input, W_hidden, W_output, b_output, bn_gamma, bn_beta):
    x = x.astype(jnp.int32)
    edge_attr = edge_attr.astype(jnp.int32)
    src = edge_index[0].astype(jnp.int32)
    dst = edge_index[1].astype(jnp.int32)
    dst_rows = dst.reshape(IDX_ROWS, 128)
    src_rows = src.reshape(IDX_ROWS, 128)
    pad28 = jnp.zeros((IDX_PAD - IDX_ROWS, 128), jnp.int32)
    pad14 = jnp.zeros(((IDX_PAD - IDX_ROWS) // NC, 128), jnp.int32)
    half = IDX_ROWS // NC
    dst_pad = jnp.concatenate([dst_rows, pad28])
    src_pad = jnp.concatenate([src_rows, pad28])
    scat_idx = jnp.concatenate(
        [dst_rows[:half], pad14, dst_rows[half:], pad14])
    zeros = jnp.zeros((NPAD, D), jnp.float32)

    h, hW = tc_encode(x, atom_tables, W_input[:D])
    g0 = sc_gather(hW, dst_pad)
    msg = tc_msg0(g0, edge_attr, bond_tables, W_input[D:])
    msg256 = msg.reshape(E2, 2 * D)

    inv_e = 1.0 / N_EDGES
    for i in range(N_LAYERS - 1):
        parts = sc_scatter(msg256.reshape(N_EDGES, D), scat_idx, zeros)
        nm = tc_sum_parts(parts)
        g = sc_gather(nm, src_pad)
        y256, stats = tc_matmul_stats(g.reshape(E2, 2 * D), msg256, W_hidden)
        s = (stats[0, :D] + stats[0, D:]) * inv_e
        sq = (stats[1, :D] + stats[1, D:]) * inv_e
        var = sq - s * s
        a = bn_gamma[i] / jnp.sqrt(var + EPS)
        b = bn_beta[i] - s * a
        a256 = jnp.concatenate([a, a])[None, :]
        b256 = jnp.concatenate([b, b])[None, :]
        msg256 = tc_affine_relu(y256, a256, b256)

    parts = sc_scatter(msg256.reshape(N_EDGES, D), scat_idx, zeros)
    return tc_final(h, parts, W_output[:D], W_output[D:], b_output[None, :])


# speedup vs baseline: 4.3343x; 1.5572x over previous
import functools
import jax
import jax.numpy as jnp
from jax import lax
from jax.experimental import pallas as pl
from jax.experimental.pallas import tpu as pltpu
from jax.experimental.pallas import tpu_sc as plsc

D = 128
N_NODES = 10000
N_EDGES = 320000
E2 = N_EDGES // 2
N_LAYERS = 5
EPS = 1e-5


def _encode_body(x_ref, at_ref, wt_ref, h_ref, hw_ref):
    xb = x_ref[...]
    acc = jnp.zeros((x_ref.shape[0], D), jnp.float32)
    for f in range(at_ref.shape[0]):
        oh = (xb[:, f][:, None] == lax.broadcasted_iota(
            jnp.int32, (x_ref.shape[0], at_ref.shape[1]), 1)).astype(jnp.float32)
        acc = acc + jnp.dot(oh, at_ref[f], preferred_element_type=jnp.float32)
    h_ref[...] = acc
    hw_ref[...] = jnp.dot(acc, wt_ref[...], preferred_element_type=jnp.float32)


def tc_encode(x, atom_tables, w_top):
    Bn = 2000
    grid = (N_NODES // Bn,)
    return pl.pallas_call(
        _encode_body,
        grid=grid,
        in_specs=[
            pl.BlockSpec((Bn, x.shape[1]), lambda i: (i, 0)),
            pl.BlockSpec(atom_tables.shape, lambda i: (0, 0, 0)),
            pl.BlockSpec((D, D), lambda i: (0, 0)),
        ],
        out_specs=[
            pl.BlockSpec((Bn, D), lambda i: (i, 0)),
            pl.BlockSpec((Bn, D), lambda i: (i, 0)),
        ],
        out_shape=[
            jax.ShapeDtypeStruct((N_NODES, D), jnp.float32),
            jax.ShapeDtypeStruct((N_NODES, D), jnp.float32),
        ],
    )(x, atom_tables, w_top)


def _msg0_body(g_ref, ea_ref, bt_ref, wb_ref, o_ref):
    ea = ea_ref[...]
    acc = jnp.zeros((ea_ref.shape[0], D), jnp.float32)
    for f in range(bt_ref.shape[0]):
        oh = (ea[:, f][:, None] == lax.broadcasted_iota(
            jnp.int32, (ea_ref.shape[0], bt_ref.shape[1]), 1)).astype(jnp.float32)
        acc = acc + jnp.dot(oh, bt_ref[f], preferred_element_type=jnp.float32)
    o_ref[...] = g_ref[...] + jnp.dot(acc, wb_ref[...],
                                      preferred_element_type=jnp.float32)


def tc_msg0(g0, edge_attr, bond_tables, w_bot):
    Be = 4000
    grid = (N_EDGES // Be,)
    return pl.pallas_call(
        _msg0_body,
        grid=grid,
        in_specs=[
            pl.BlockSpec((Be, D), lambda i: (i, 0)),
            pl.BlockSpec((Be, edge_attr.shape[1]), lambda i: (i, 0)),
            pl.BlockSpec(bond_tables.shape, lambda i: (0, 0, 0)),
            pl.BlockSpec((D, D), lambda i: (0, 0)),
        ],
        out_specs=pl.BlockSpec((Be, D), lambda i: (i, 0)),
        out_shape=jax.ShapeDtypeStruct((N_EDGES, D), jnp.float32),
    )(g0, edge_attr, bond_tables, w_bot)


def _sum_parts_body(p_ref, o_ref):
    o_ref[...] = p_ref[0] + p_ref[1]


def tc_sum_parts(parts):
    Bn = 2000
    grid = (N_NODES // Bn,)
    return pl.pallas_call(
        _sum_parts_body,
        grid=grid,
        in_specs=[pl.BlockSpec((2, Bn, D), lambda i: (0, i, 0))],
        out_specs=pl.BlockSpec((Bn, D), lambda i: (i, 0)),
        out_shape=jax.ShapeDtypeStruct((N_NODES, D), jnp.float32),
    )(parts)


def _mm_stats_body(g_ref, m_ref, w_ref, y_ref, s_ref):
    g = g_ref[...]
    m = m_ref[...]
    up = jnp.concatenate([m[1:], m[:1]], axis=0)
    dn = jnp.concatenate([m[-1:], m[:-1]], axis=0)
    even = (lax.broadcasted_iota(jnp.int32, m.shape, 0) % 2) == 0
    u = g - jnp.where(even, up, dn)
    y = jnp.dot(u, w_ref[...], preferred_element_type=jnp.float32)
    y_ref[...] = y

    @pl.when(pl.program_id(0) == 0)
    def _():
        s_ref[...] = jnp.zeros_like(s_ref)

    s_ref[0, :] += jnp.sum(y, axis=0)
    s_ref[1, :] += jnp.sum(y * y, axis=0)


def tc_matmul_stats(g, msg, w_hidden):
    Be = 6400
    grid = (N_EDGES // Be,)
    return pl.pallas_call(
        _mm_stats_body,
        grid=grid,
        in_specs=[
            pl.BlockSpec((Be, D), lambda i: (i, 0)),
            pl.BlockSpec((Be, D), lambda i: (i, 0)),
            pl.BlockSpec((D, D), lambda i: (0, 0)),
        ],
        out_specs=[
            pl.BlockSpec((Be, D), lambda i: (i, 0)),
            pl.BlockSpec((8, D), lambda i: (0, 0)),
        ],
        out_shape=[
            jax.ShapeDtypeStruct((N_EDGES, D), jnp.float32),
            jax.ShapeDtypeStruct((8, D), jnp.float32),
        ],
    )(g, msg, w_hidden)


def _affine_relu_body(y_ref, a_ref, b_ref, o_ref):
    o_ref[...] = jnp.maximum(y_ref[...] * a_ref[...] + b_ref[...], 0.0)


def tc_affine_relu(y, a, b):
    Be = 8000
    grid = (N_EDGES // Be,)
    return pl.pallas_call(
        _affine_relu_body,
        grid=grid,
        in_specs=[
            pl.BlockSpec((Be, D), lambda i: (i, 0)),
            pl.BlockSpec((1, D), lambda i: (0, 0)),
            pl.BlockSpec((1, D), lambda i: (0, 0)),
        ],
        out_specs=pl.BlockSpec((Be, D), lambda i: (i, 0)),
        out_shape=jax.ShapeDtypeStruct((N_EDGES, D), jnp.float32),
    )(y, a, b)


def _final_body(h_ref, p_ref, wt_ref, wb_ref, b_ref, o_ref):
    nm = p_ref[0] + p_ref[1]
    r = (jnp.dot(h_ref[...], wt_ref[...], preferred_element_type=jnp.float32)
         + jnp.dot(nm, wb_ref[...], preferred_element_type=jnp.float32)
         + b_ref[...])
    o_ref[...] = jnp.maximum(r, 0.0)


def tc_final(h, parts, w_top, w_bot, bias):
    Bn = 2000
    grid = (N_NODES // Bn,)
    return pl.pallas_call(
        _final_body,
        grid=grid,
        in_specs=[
            pl.BlockSpec((Bn, D), lambda i: (i, 0)),
            pl.BlockSpec((2, Bn, D), lambda i: (0, i, 0)),
            pl.BlockSpec((D, D), lambda i: (0, 0)),
            pl.BlockSpec((D, D), lambda i: (0, 0)),
            pl.BlockSpec((1, D), lambda i: (0, 0)),
        ],
        out_specs=pl.BlockSpec((Bn, D), lambda i: (i, 0)),
        out_shape=jax.ShapeDtypeStruct((N_NODES, D), jnp.float32),
    )(h, parts, w_top, w_bot, bias)


NC = 2
NS = 16
NW = NC * NS
IDX_ROWS = N_EDGES // 128
CPW = 80
IDX_PAD = NW * CPW
NPAD = 10112


def sc_gather(table, idx_rows):
    mesh = plsc.VectorSubcoreMesh(core_axis_name="c", subcore_axis_name="s")

    @functools.partial(
        pl.kernel, mesh=mesh,
        out_type=jax.ShapeDtypeStruct((N_EDGES, D), jnp.float32),
        scratch_types=[
            pltpu.VMEM((CPW, 128), jnp.int32),
            pltpu.VMEM((2, 128, D), jnp.float32),
            pltpu.SemaphoreType.DMA((2,)),
        ],
    )
    def k(table_hbm, idx_hbm, out_hbm, idx_v, rows_v, gsem):
        c = lax.axis_index("c")
        s = lax.axis_index("s")
        wid = s * NC + c
        lo = wid * CPW
        pltpu.sync_copy(idx_hbm.at[pl.ds(lo, CPW)], idx_v)

        def start(t, b):
            @pl.when(lo + t < IDX_ROWS)
            def _():
                pltpu.async_copy(table_hbm.at[idx_v.at[t]], rows_v.at[b],
                                 gsem.at[b])

        def finish(t, b):
            @pl.when(lo + t < IDX_ROWS)
            def _():
                pltpu.make_async_copy(table_hbm.at[idx_v.at[t]], rows_v.at[b],
                                      gsem.at[b]).wait()
                pltpu.sync_copy(rows_v.at[b], out_hbm.at[pl.ds((lo + t) * 128, 128)])

        start(0, 0)
        start(1, 1)

        @pl.loop(0, CPW // 2)
        def _(i):
            t = i * 2
            finish(t, 0)

            @pl.when(t + 2 < CPW)
            def _():
                start(t + 2, 0)

            finish(t + 1, 1)

            @pl.when(t + 3 < CPW)
            def _():
                start(t + 3, 1)

    return k(table, idx_rows)


def sc_scatter(msg, scat_idx, zeros):
    mesh = plsc.VectorSubcoreMesh(core_axis_name="c", subcore_axis_name="s")
    rows_per_core = IDX_ROWS // NC
    core_pad = NS * CPW
    stripe = NPAD // NS

    @functools.partial(
        pl.kernel, mesh=mesh,
        out_type=jax.ShapeDtypeStruct((NC, NPAD, D), jnp.float32),
        scratch_types=[
            pltpu.VMEM((CPW, 128), jnp.int32),
            pltpu.VMEM((2, 128, D), jnp.float32),
            pltpu.VMEM_SHARED((NPAD, D), jnp.float32),
            pltpu.SemaphoreType.DMA((2,)),
        ],
    )
    def k(msg_hbm, idx_hbm, z_hbm, out_hbm, idx_v, rows_v, acc_sh, lsem):
        c = lax.axis_index("c")
        s = lax.axis_index("s")
        pltpu.sync_copy(z_hbm.at[pl.ds(s * stripe, stripe)],
                        acc_sh.at[pl.ds(s * stripe, stripe)])
        base_l = s * CPW
        pltpu.sync_copy(idx_hbm.at[pl.ds(c * core_pad + base_l, CPW)], idx_v)
        plsc.subcore_barrier()

        def start(u, b):
            @pl.when(base_l + u < rows_per_core)
            def _():
                pltpu.async_copy(
                    msg_hbm.at[pl.ds((c * rows_per_core + base_l + u) * 128, 128)],
                    rows_v.at[b], lsem.at[b])

        def finish(u, b):
            @pl.when(base_l + u < rows_per_core)
            def _():
                pltpu.make_async_copy(
                    msg_hbm.at[pl.ds((c * rows_per_core + base_l + u) * 128, 128)],
                    rows_v.at[b], lsem.at[b]).wait()
                pltpu.sync_copy(rows_v.at[b], acc_sh.at[idx_v.at[u]], add=True)

        start(0, 0)
        start(1, 1)

        @pl.loop(0, CPW // 2)
        def _(i):
            u = i * 2
            finish(u, 0)

            @pl.when(u + 2 < CPW)
            def _():
                start(u + 2, 0)

            finish(u + 1, 1)

            @pl.when(u + 3 < CPW)
            def _():
                start(u + 3, 1)

        plsc.subcore_barrier()
        pltpu.sync_copy(acc_sh.at[pl.ds(s * stripe, stripe)],
                        out_hbm.at[c, pl.ds(s * stripe, stripe)])

    return k(msg, scat_idx, zeros)


def kernel(x, edge_index, edge_attr, atom_tables, bond_tables,
           W_input, W_hidden, W_output, b_output, bn_gamma, bn_beta):
    x = x.astype(jnp.int32)
    edge_attr = edge_attr.astype(jnp.int32)
    src = edge_index[0].astype(jnp.int32)
    dst = edge_index[1].astype(jnp.int32)
    dst_rows = dst.reshape(IDX_ROWS, 128)
    src_rows = src.reshape(IDX_ROWS, 128)
    pad28 = jnp.zeros((IDX_PAD - IDX_ROWS, 128), jnp.int32)
    pad14 = jnp.zeros(((IDX_PAD - IDX_ROWS) // NC, 128), jnp.int32)
    half = IDX_ROWS // NC
    dst_pad = jnp.concatenate([dst_rows, pad28])
    src_pad = jnp.concatenate([src_rows, pad28])
    scat_idx = jnp.concatenate(
        [dst_rows[:half], pad14, dst_rows[half:], pad14])
    zeros = jnp.zeros((NPAD, D), jnp.float32)

    h, hW = tc_encode(x, atom_tables, W_input[:D])
    g0 = sc_gather(hW, dst_pad)
    msg = tc_msg0(g0, edge_attr, bond_tables, W_input[D:])

    inv_e = 1.0 / N_EDGES
    for i in range(N_LAYERS - 1):
        parts = sc_scatter(msg, scat_idx, zeros)
        nm = tc_sum_parts(parts)
        g = sc_gather(nm, src_pad)
        y, stats = tc_matmul_stats(g, msg, W_hidden)
        s = stats[0] * inv_e
        var = stats[1] * inv_e - s * s
        a = bn_gamma[i] / jnp.sqrt(var + EPS)
        b = bn_beta[i] - s * a
        msg = tc_affine_relu(y, a[None, :], b[None, :])

    parts = sc_scatter(msg, scat_idx, zeros)
    return tc_final(h, parts, W_output[:D], W_output[D:], b_output[None, :])


# speedup vs baseline: 4.3479x; 1.0031x over previous
import functools
import jax
import jax.numpy as jnp
from jax import lax
from jax.experimental import pallas as pl
from jax.experimental.pallas import tpu as pltpu
from jax.experimental.pallas import tpu_sc as plsc

D = 128
N_NODES = 10000
N_EDGES = 320000
E2 = N_EDGES // 2
N_LAYERS = 5
EPS = 1e-5


def _encode_body(x_ref, at_ref, wt_ref, h_ref, hw_ref):
    xb = x_ref[...]
    acc = jnp.zeros((x_ref.shape[0], D), jnp.float32)
    for f in range(at_ref.shape[0]):
        oh = (xb[:, f][:, None] == lax.broadcasted_iota(
            jnp.int32, (x_ref.shape[0], at_ref.shape[1]), 1)).astype(jnp.float32)
        acc = acc + jnp.dot(oh, at_ref[f], preferred_element_type=jnp.float32)
    h_ref[...] = acc
    hw_ref[...] = jnp.dot(acc, wt_ref[...], preferred_element_type=jnp.float32)


def tc_encode(x, atom_tables, w_top):
    Bn = 2000
    grid = (N_NODES // Bn,)
    return pl.pallas_call(
        _encode_body,
        grid=grid,
        in_specs=[
            pl.BlockSpec((Bn, x.shape[1]), lambda i: (i, 0)),
            pl.BlockSpec(atom_tables.shape, lambda i: (0, 0, 0)),
            pl.BlockSpec((D, D), lambda i: (0, 0)),
        ],
        out_specs=[
            pl.BlockSpec((Bn, D), lambda i: (i, 0)),
            pl.BlockSpec((Bn, D), lambda i: (i, 0)),
        ],
        out_shape=[
            jax.ShapeDtypeStruct((N_NODES, D), jnp.float32),
            jax.ShapeDtypeStruct((N_NODES, D), jnp.float32),
        ],
    )(x, atom_tables, w_top)


def _msg0_body(g_ref, ea_ref, bt_ref, wb_ref, o_ref):
    ea = ea_ref[...]
    acc = jnp.zeros((ea_ref.shape[0], D), jnp.float32)
    for f in range(bt_ref.shape[0]):
        oh = (ea[:, f][:, None] == lax.broadcasted_iota(
            jnp.int32, (ea_ref.shape[0], bt_ref.shape[1]), 1)).astype(jnp.float32)
        acc = acc + jnp.dot(oh, bt_ref[f], preferred_element_type=jnp.float32)
    o_ref[...] = g_ref[...] + jnp.dot(acc, wb_ref[...],
                                      preferred_element_type=jnp.float32)


def tc_msg0(g0, edge_attr, bond_tables, w_bot):
    Be = 4000
    grid = (N_EDGES // Be,)
    return pl.pallas_call(
        _msg0_body,
        grid=grid,
        in_specs=[
            pl.BlockSpec((Be, D), lambda i: (i, 0)),
            pl.BlockSpec((Be, edge_attr.shape[1]), lambda i: (i, 0)),
            pl.BlockSpec(bond_tables.shape, lambda i: (0, 0, 0)),
            pl.BlockSpec((D, D), lambda i: (0, 0)),
        ],
        out_specs=pl.BlockSpec((Be, D), lambda i: (i, 0)),
        out_shape=jax.ShapeDtypeStruct((N_EDGES, D), jnp.float32),
    )(g0, edge_attr, bond_tables, w_bot)


def _sum_parts_body(p_ref, o_ref):
    o_ref[...] = p_ref[0] + p_ref[1]


def tc_sum_parts(parts):
    Bn = 2000
    grid = (N_NODES // Bn,)
    return pl.pallas_call(
        _sum_parts_body,
        grid=grid,
        in_specs=[pl.BlockSpec((2, Bn, D), lambda i: (0, i, 0))],
        out_specs=pl.BlockSpec((Bn, D), lambda i: (i, 0)),
        out_shape=jax.ShapeDtypeStruct((N_NODES, D), jnp.float32),
    )(parts)


def _mm_stats_body(g_ref, m_ref, w_ref, y_ref, s_ref):
    g = g_ref[...]
    m = m_ref[...]
    up = jnp.concatenate([m[1:], m[:1]], axis=0)
    dn = jnp.concatenate([m[-1:], m[:-1]], axis=0)
    even = (lax.broadcasted_iota(jnp.int32, m.shape, 0) % 2) == 0
    u = g - jnp.where(even, up, dn)
    y = jnp.dot(u, w_ref[...], preferred_element_type=jnp.float32)
    y_ref[...] = y

    @pl.when(pl.program_id(0) == 0)
    def _():
        s_ref[...] = jnp.zeros_like(s_ref)

    s_ref[0, :] += jnp.sum(y, axis=0)
    s_ref[1, :] += jnp.sum(y * y, axis=0)


def tc_matmul_stats(g, msg, w_hidden):
    Be = 6400
    grid = (N_EDGES // Be,)
    return pl.pallas_call(
        _mm_stats_body,
        grid=grid,
        in_specs=[
            pl.BlockSpec((Be, D), lambda i: (i, 0)),
            pl.BlockSpec((Be, D), lambda i: (i, 0)),
            pl.BlockSpec((D, D), lambda i: (0, 0)),
        ],
        out_specs=[
            pl.BlockSpec((Be, D), lambda i: (i, 0)),
            pl.BlockSpec((8, D), lambda i: (0, 0)),
        ],
        out_shape=[
            jax.ShapeDtypeStruct((N_EDGES, D), jnp.float32),
            jax.ShapeDtypeStruct((8, D), jnp.float32),
        ],
    )(g, msg, w_hidden)


def _affine_relu_body(y_ref, a_ref, b_ref, o_ref):
    o_ref[...] = jnp.maximum(y_ref[...] * a_ref[...] + b_ref[...], 0.0)


def tc_affine_relu(y, a, b):
    Be = 8000
    grid = (N_EDGES // Be,)
    return pl.pallas_call(
        _affine_relu_body,
        grid=grid,
        in_specs=[
            pl.BlockSpec((Be, D), lambda i: (i, 0)),
            pl.BlockSpec((1, D), lambda i: (0, 0)),
            pl.BlockSpec((1, D), lambda i: (0, 0)),
        ],
        out_specs=pl.BlockSpec((Be, D), lambda i: (i, 0)),
        out_shape=jax.ShapeDtypeStruct((N_EDGES, D), jnp.float32),
    )(y, a, b)


def _final_body(h_ref, p_ref, wt_ref, wb_ref, b_ref, o_ref):
    nm = p_ref[0] + p_ref[1]
    r = (jnp.dot(h_ref[...], wt_ref[...], preferred_element_type=jnp.float32)
         + jnp.dot(nm, wb_ref[...], preferred_element_type=jnp.float32)
         + b_ref[...])
    o_ref[...] = jnp.maximum(r, 0.0)


def tc_final(h, parts, w_top, w_bot, bias):
    Bn = 2000
    grid = (N_NODES // Bn,)
    return pl.pallas_call(
        _final_body,
        grid=grid,
        in_specs=[
            pl.BlockSpec((Bn, D), lambda i: (i, 0)),
            pl.BlockSpec((2, Bn, D), lambda i: (0, i, 0)),
            pl.BlockSpec((D, D), lambda i: (0, 0)),
            pl.BlockSpec((D, D), lambda i: (0, 0)),
            pl.BlockSpec((1, D), lambda i: (0, 0)),
        ],
        out_specs=pl.BlockSpec((Bn, D), lambda i: (i, 0)),
        out_shape=jax.ShapeDtypeStruct((N_NODES, D), jnp.float32),
    )(h, parts, w_top, w_bot, bias)


NC = 2
NS = 16
NW = NC * NS
IDX_ROWS = N_EDGES // 128
CPW = 80
IDX_PAD = NW * CPW
NPAD = 10112


def sc_gather(table, idx_rows):
    mesh = plsc.VectorSubcoreMesh(core_axis_name="c", subcore_axis_name="s")

    @functools.partial(
        pl.kernel, mesh=mesh,
        out_type=jax.ShapeDtypeStruct((N_EDGES, D), jnp.float32),
        scratch_types=[
            pltpu.VMEM((CPW, 128), jnp.int32),
            pltpu.VMEM((4, 128, D), jnp.float32),
            pltpu.SemaphoreType.DMA((4,)),
            pltpu.SemaphoreType.DMA((4,)),
        ],
    )
    def k(table_hbm, idx_hbm, out_hbm, idx_v, rows_v, gsem, ssem):
        c = lax.axis_index("c")
        s = lax.axis_index("s")
        wid = s * NC + c
        lo = wid * CPW
        pltpu.sync_copy(idx_hbm.at[pl.ds(lo, CPW)], idx_v)

        def gath(t, b):
            @pl.when((t >= 0) & (t < CPW) & (lo + t < IDX_ROWS))
            def _():
                pltpu.async_copy(table_hbm.at[idx_v.at[t]], rows_v.at[b],
                                 gsem.at[b])

        def store(t, b):
            @pl.when((lo + t < IDX_ROWS))
            def _():
                pltpu.make_async_copy(table_hbm.at[idx_v.at[t]], rows_v.at[b],
                                      gsem.at[b]).wait()
                pltpu.async_copy(rows_v.at[b],
                                 out_hbm.at[pl.ds((lo + t) * 128, 128)],
                                 ssem.at[b])

        def drain(t, b):
            @pl.when((t >= 0) & (lo + t < IDX_ROWS))
            def _():
                pltpu.make_async_copy(rows_v.at[b],
                                      out_hbm.at[pl.ds((lo + t) * 128, 128)],
                                      ssem.at[b]).wait()

        gath(0, 0)
        gath(1, 1)

        @pl.loop(0, CPW // 4)
        def _(i):
            for kk in range(4):
                t = i * 4 + kk
                store(t, kk)
                drain(t - 2, (kk + 2) % 4)
                gath(t + 2, (kk + 2) % 4)

        drain(CPW - 2, (CPW - 2) % 4)
        drain(CPW - 1, (CPW - 1) % 4)

    return k(table, idx_rows)


def sc_scatter(msg, scat_idx, zeros):
    mesh = plsc.VectorSubcoreMesh(core_axis_name="c", subcore_axis_name="s")
    rows_per_core = IDX_ROWS // NC
    core_pad = NS * CPW
    stripe = NPAD // NS

    @functools.partial(
        pl.kernel, mesh=mesh,
        out_type=jax.ShapeDtypeStruct((NC, NPAD, D), jnp.float32),
        scratch_types=[
            pltpu.VMEM((CPW, 128), jnp.int32),
            pltpu.VMEM((2, 128, D), jnp.float32),
            pltpu.VMEM_SHARED((NPAD, D), jnp.float32),
            pltpu.SemaphoreType.DMA((2,)),
        ],
    )
    def k(msg_hbm, idx_hbm, z_hbm, out_hbm, idx_v, rows_v, acc_sh, lsem):
        c = lax.axis_index("c")
        s = lax.axis_index("s")
        pltpu.sync_copy(z_hbm.at[pl.ds(s * stripe, stripe)],
                        acc_sh.at[pl.ds(s * stripe, stripe)])
        base_l = s * CPW
        pltpu.sync_copy(idx_hbm.at[pl.ds(c * core_pad + base_l, CPW)], idx_v)
        plsc.subcore_barrier()

        def load(u, b):
            @pl.when((u >= 0) & (u < CPW) & (base_l + u < rows_per_core))
            def _():
                pltpu.async_copy(
                    msg_hbm.at[pl.ds((c * rows_per_core + base_l + u) * 128, 128)],
                    rows_v.at[b], lsem.at[b])

        def add(u, b):
            @pl.when(base_l + u < rows_per_core)
            def _():
                pltpu.make_async_copy(
                    msg_hbm.at[pl.ds((c * rows_per_core + base_l + u) * 128, 128)],
                    rows_v.at[b], lsem.at[b]).wait()
                pltpu.sync_copy(rows_v.at[b], acc_sh.at[idx_v.at[u]], add=True)

        load(0, 0)
        load(1, 1)

        @pl.loop(0, CPW // 2)
        def _(i):
            for kk in range(2):
                u = i * 2 + kk
                add(u, kk)
                load(u + 2, kk)

        plsc.subcore_barrier()
        pltpu.sync_copy(acc_sh.at[pl.ds(s * stripe, stripe)],
                        out_hbm.at[c, pl.ds(s * stripe, stripe)])

    return k(msg, scat_idx, zeros)


def kernel(x, edge_index, edge_attr, atom_tables, bond_tables,
           W_input, W_hidden, W_output, b_output, bn_gamma, bn_beta):
    x = x.astype(jnp.int32)
    edge_attr = edge_attr.astype(jnp.int32)
    src = edge_index[0].astype(jnp.int32)
    dst = edge_index[1].astype(jnp.int32)
    dst_rows = dst.reshape(IDX_ROWS, 128)
    src_rows = src.reshape(IDX_ROWS, 128)
    pad28 = jnp.zeros((IDX_PAD - IDX_ROWS, 128), jnp.int32)
    pad14 = jnp.zeros(((IDX_PAD - IDX_ROWS) // NC, 128), jnp.int32)
    half = IDX_ROWS // NC
    dst_pad = jnp.concatenate([dst_rows, pad28])
    src_pad = jnp.concatenate([src_rows, pad28])
    scat_idx = jnp.concatenate(
        [dst_rows[:half], pad14, dst_rows[half:], pad14])
    zeros = jnp.zeros((NPAD, D), jnp.float32)

    h, hW = tc_encode(x, atom_tables, W_input[:D])
    g0 = sc_gather(hW, dst_pad)
    msg = tc_msg0(g0, edge_attr, bond_tables, W_input[D:])

    inv_e = 1.0 / N_EDGES
    for i in range(N_LAYERS - 1):
        parts = sc_scatter(msg, scat_idx, zeros)
        nm = tc_sum_parts(parts)
        g = sc_gather(nm, src_pad)
        y, stats = tc_matmul_stats(g, msg, W_hidden)
        s = stats[0] * inv_e
        var = stats[1] * inv_e - s * s
        a = bn_gamma[i] / jnp.sqrt(var + EPS)
        b = bn_beta[i] - s * a
        msg = tc_affine_relu(y, a[None, :], b[None, :])

    parts = sc_scatter(msg, scat_idx, zeros)
    return tc_final(h, parts, W_output[:D], W_output[D:], b_output[None, :])
